# baseline (speedup 1.0000x reference)
import jax
import jax.numpy as jnp
from jax import lax
from jax.experimental import pallas as pl
from jax.experimental.pallas import tpu as pltpu
from jax.experimental.pallas import tpu_sc as plsc

B, Np1, D, K = 8, 2048, 128, 16
NW = 32
NBLK = 4
DB = D // NBLK
G = 16
NG = Np1 // G


def _sc_body(psiT_hbm, idxT_hbm, qb_hbm, ctx_hbm, intf_hbm,
             tbl_v, idx_v, qb_v, ctx_v, intf_v):
    cid = lax.axis_index("c")
    sid = lax.axis_index("s")
    wid = sid * 2 + cid
    b = wid // NBLK
    pltpu.sync_copy(psiT_hbm.at[wid], tbl_v)
    pltpu.sync_copy(idxT_hbm.at[b], idx_v)
    pltpu.sync_copy(qb_hbm.at[wid], qb_v)

    def group(g, carry):
        sl = pl.ds(g * G, G)
        idxs = [idx_v[k, sl] for k in range(K)]
        iv = jnp.zeros((G,), jnp.float32)
        cv = jnp.zeros((G,), jnp.float32)
        for d in range(DB):
            dfull = jnp.full((G,), d, jnp.int32)
            vals = [plsc.load_gather(tbl_v, [dfull, idxs[k]]) for k in range(K)]
            while len(vals) > 1:
                vals = [vals[2 * m] + vals[2 * m + 1]
                        for m in range(len(vals) // 2)]
            own = tbl_v[d, sl]
            iv = iv + own * vals[0]
            cv = cv + own * qb_v[d]
        ctx_v[sl] = cv
        intf_v[sl] = iv
        return carry

    lax.fori_loop(0, NG, group, 0)
    pltpu.sync_copy(ctx_v, ctx_hbm.at[wid])
    pltpu.sync_copy(intf_v, intf_hbm.at[wid])


def _make_sc_pool():
    return pl.kernel(
        _sc_body,
        out_type=[
            jax.ShapeDtypeStruct((NW, Np1), jnp.float32),
            jax.ShapeDtypeStruct((NW, Np1), jnp.float32),
        ],
        mesh=plsc.VectorSubcoreMesh(core_axis_name="c", subcore_axis_name="s"),
        compiler_params=pltpu.CompilerParams(needs_layout_passes=False),
        scratch_types=[
            pltpu.VMEM((DB, Np1), jnp.float32),
            pltpu.VMEM((K, Np1), jnp.int32),
            pltpu.VMEM((DB, G), jnp.float32),
            pltpu.VMEM((Np1,), jnp.float32),
            pltpu.VMEM((Np1,), jnp.float32),
        ],
    )


_sc_cache = []


def _sc_pool(psiT, idxT, qb):
    if not _sc_cache:
        _sc_cache.append(_make_sc_pool())
    return _sc_cache[0](psiT, idxT, qb)


def _tc_body(ctxp_ref, intfp_ref, ax_ref, cur_ref, maskf_ref, par_ref,
             out_ref):
    lam = jnp.clip(par_ref[0], -0.5, 3.0)
    mu = jnp.clip(par_ref[1], 0.0, 10.0)
    ctx = jnp.sum(ctxp_ref[...], axis=1)
    intf = jnp.sum(intfp_ref[...], axis=1)
    d0 = ax_ref[0] - cur_ref[0]
    d1 = ax_ref[1] - cur_ref[1]
    dist = jnp.sqrt(d0 * d0 + d1 * d1)
    s = ctx + lam * intf - mu * dist
    s = jnp.where(maskf_ref[...] != 0.0, -1000000000.0, s)
    m = jnp.max(s, axis=1, keepdims=True)
    ls = s - m
    z = jnp.sum(jnp.exp(ls), axis=1, keepdims=True)
    out_ref[...] = ls - jnp.log(z)


def kernel(query, psi_prime, knn_indices, mask, current_coords, all_coords,
           lambda_param, mu_param):
    psiT = (psi_prime.reshape(B, Np1, NBLK, DB)
            .transpose(0, 2, 3, 1)
            .reshape(NW, DB, Np1))
    idxT = knn_indices.astype(jnp.int32).transpose(0, 2, 1)
    qb = jnp.broadcast_to(query.reshape(NW, DB, 1), (NW, DB, G))

    ctxp, intfp = _sc_pool(psiT, idxT, qb)
    ctxp = ctxp.reshape(B, NBLK, Np1)
    intfp = intfp.reshape(B, NBLK, Np1)

    ax_t = all_coords.transpose(2, 0, 1)
    cur_t = current_coords.T[:, :, None]
    maskf = mask.astype(jnp.float32)
    par = jnp.stack([lambda_param.astype(jnp.float32),
                     mu_param.astype(jnp.float32)])

    return pl.pallas_call(
        _tc_body,
        out_shape=jax.ShapeDtypeStruct((B, Np1), jnp.float32),
        in_specs=[
            pl.BlockSpec(memory_space=pltpu.VMEM),
            pl.BlockSpec(memory_space=pltpu.VMEM),
            pl.BlockSpec(memory_space=pltpu.VMEM),
            pl.BlockSpec(memory_space=pltpu.VMEM),
            pl.BlockSpec(memory_space=pltpu.VMEM),
            pl.BlockSpec(memory_space=pltpu.SMEM),
        ],
        out_specs=pl.BlockSpec(memory_space=pltpu.VMEM),
    )(ctxp, intfp, ax_t, cur_t, maskf, par)
